# baseline (speedup 1.0000x reference)
import functools

import jax
import jax.numpy as jnp
from jax import lax
from jax.experimental import pallas as pl
from jax.experimental.pallas import tpu as pltpu
from jax.experimental.pallas import tpu_sc as plsc

K = 8
EPS = 0.1
RB = 256
EF_ROWS = 4224
WC = 128
F_TC = 0
NCH = (4096 - F_TC) // WC
CPC = NCH // 2
TPN = 256
SLABS = 16
JB = EF_ROWS // 16


def _normalize_body(x_ref, o_ref):
    x = x_ref[...]
    nrm = jnp.sqrt(jnp.sum(x * x, axis=1, keepdims=True))
    o_ref[...] = x / jnp.maximum(nrm, 1e-12)


def _graph_body(xb_ref, xall_ref, idx_ref, w_ref, dinv_ref, idxsg_ref):
    n = xall_ref.shape[0]
    i0 = pl.program_id(0)
    s = lax.dot_general(xb_ref[...], xall_ref[...], (((1,), (1,)), ((), ())),
                        preferred_element_type=jnp.float32)
    cols = lax.broadcasted_iota(jnp.int32, (RB, n), 1)
    kcols = lax.broadcasted_iota(jnp.int32, (RB, K), 1)
    rows = lax.broadcasted_iota(jnp.int32, (RB, K), 0) + i0 * RB
    idx_out = jnp.zeros((RB, K), jnp.int32)
    w_out = jnp.zeros((RB, K), jnp.float32)
    for k in range(K):
        m = jnp.max(s, axis=1, keepdims=True)
        amin = jnp.min(jnp.where(s == m, cols, n), axis=1, keepdims=True)
        idx_out = jnp.where(kcols == k, amin, idx_out)
        w_out = jnp.where(kcols == k, (m > EPS).astype(jnp.float32), w_out)
        s = jnp.where(cols == amin, -jnp.inf, s)
    ddeg = jnp.sum(w_out, axis=1, keepdims=True)
    dinv = jnp.where(ddeg > 0, 1.0 / ddeg, 0.0)
    dummy = 4096 + (rows * K + kcols) % 128
    idx_ref[...] = idx_out
    w_ref[...] = w_out
    dinv_ref[...] = jnp.broadcast_to(dinv, (RB, K))
    idxsg_ref[...] = jnp.where(w_out > 0, idx_out, dummy)


def _matmul_body(a_ref, b_ref, o_ref):
    o_ref[...] = jnp.dot(a_ref[...], b_ref[...],
                         preferred_element_type=jnp.float32)


def _matmul_bf16_body(a_ref, b_ref, o_ref):
    a = a_ref[...].astype(jnp.bfloat16)
    b = b_ref[...].astype(jnp.bfloat16)
    o_ref[...] = jnp.dot(a, b, preferred_element_type=jnp.float32)


def _binv_body(idxT_ref, wT_ref, o_ref):
    j = pl.program_id(0)
    ji = j * JB + lax.broadcasted_iota(jnp.int32, (JB, 1), 0)
    bdeg = jnp.zeros((JB, 1), jnp.float32)
    for k in range(K):
        rowi = idxT_ref[k:k + 1, :]
        roww = wT_ref[k:k + 1, :]
        bdeg = bdeg + jnp.sum(jnp.where(rowi == ji, roww, 0.0),
                              axis=1, keepdims=True)
    binv = jnp.where(bdeg > 0, 1.0 / bdeg, 0.0)
    o_ref[...] = jnp.broadcast_to(binv, (JB, 128))


def _scale_body(ef_ref, binv_ref, o_ref):
    ef = ef_ref[...].reshape(EF_ROWS, WC)
    b = binv_ref[:, 0:1]
    o_ref[...] = jnp.where(b > 0, ef * b, 0.0).reshape(1, EF_ROWS, WC)


def _xl2_body(a_ref, dinv_ref, b1_ref, th2_ref, o_ref):
    a = jax.nn.relu(a_ref[...] * dinv_ref[:, 0:1] + b1_ref[...])
    o_ref[...] = jnp.dot(a, th2_ref[...], preferred_element_type=jnp.float32)


def _edge_body(idxT_ref, wT_ref, xl_ref, ef_ref):
    n = xl_ref.shape[0]
    j = pl.program_id(1)
    ji = j * RB + lax.broadcasted_iota(jnp.int32, (RB, 1), 0)
    ht = jnp.zeros((RB, n), jnp.float32)
    for k in range(K):
        rowi = idxT_ref[k:k + 1, :]
        roww = wT_ref[k:k + 1, :]
        ht = ht + jnp.where(rowi == ji, roww, 0.0)
    bdeg = jnp.sum(ht, axis=1, keepdims=True)
    binv = jnp.where(bdeg > 0, 1.0 / bdeg, 0.0)
    ef_ref[...] = jnp.dot(ht, xl_ref[...],
                          preferred_element_type=jnp.float32) * binv


def _node_body(idx_ref, w_ref, ef_ref, b_ref, o_ref, *, act):
    n = ef_ref.shape[0]
    cols = lax.broadcasted_iota(jnp.int32, (RB, n), 1)
    h = jnp.zeros((RB, n), jnp.float32)
    for k in range(K):
        h = h + jnp.where(idx_ref[:, k:k + 1] == cols, w_ref[:, k:k + 1], 0.0)
    ddeg = jnp.sum(w_ref[...], axis=1, keepdims=True)
    dinv = jnp.where(ddeg > 0, 1.0 / ddeg, 0.0)
    out = jnp.dot(h, ef_ref[...],
                  preferred_element_type=jnp.float32) * dinv + b_ref[...]
    o_ref[...] = act(out)


def _node_raw_body(idx_ref, w_ref, ef_ref, o_ref):
    n = ef_ref.shape[0]
    cols = lax.broadcasted_iota(jnp.int32, (RB, n), 1)
    h = jnp.zeros((RB, n), jnp.float32)
    for k in range(K):
        h = h + jnp.where(idx_ref[:, k:k + 1] == cols, w_ref[:, k:k + 1], 0.0)
    o_ref[...] = jnp.dot(h, ef_ref[...], preferred_element_type=jnp.float32)


def _node_raw(idx, w, ef, cb):
    n = ef.shape[0]
    w_ = ef.shape[1]
    return pl.pallas_call(
        _node_raw_body,
        grid=(w_ // cb, n // RB),
        in_specs=[pl.BlockSpec((RB, K), lambda c, i: (i, 0)),
                  pl.BlockSpec((RB, K), lambda c, i: (i, 0)),
                  pl.BlockSpec((n, cb), lambda c, i: (0, c))],
        out_specs=pl.BlockSpec((RB, cb), lambda c, i: (i, c)),
        out_shape=jax.ShapeDtypeStruct((n, w_), jnp.float32),
    )(idx, w, ef)


def _matmul(a, b, cb, body=_matmul_body):
    m, kd = a.shape
    _, nd = b.shape
    return pl.pallas_call(
        body,
        grid=(nd // cb, m // RB),
        in_specs=[pl.BlockSpec((RB, kd), lambda c, i: (i, 0)),
                  pl.BlockSpec((kd, cb), lambda c, i: (0, c))],
        out_specs=pl.BlockSpec((RB, cb), lambda c, i: (i, c)),
        out_shape=jax.ShapeDtypeStruct((m, nd), jnp.float32),
    )(a, b)


def _edge_agg(idxT, wT, xl, cb):
    n = xl.shape[0]
    w_ = xl.shape[1]
    return pl.pallas_call(
        _edge_body,
        grid=(w_ // cb, n // RB),
        in_specs=[pl.BlockSpec((K, n), lambda c, j: (0, 0)),
                  pl.BlockSpec((K, n), lambda c, j: (0, 0)),
                  pl.BlockSpec((n, cb), lambda c, j: (0, c))],
        out_specs=pl.BlockSpec((RB, cb), lambda c, j: (j, c)),
        out_shape=jax.ShapeDtypeStruct((n, w_), jnp.float32),
    )(idxT, wT, xl)


def _node_agg(idx, w, ef, bias, cb, act):
    n = ef.shape[0]
    w_ = ef.shape[1]
    return pl.pallas_call(
        functools.partial(_node_body, act=act),
        grid=(w_ // cb, n // RB),
        in_specs=[pl.BlockSpec((RB, K), lambda c, i: (i, 0)),
                  pl.BlockSpec((RB, K), lambda c, i: (i, 0)),
                  pl.BlockSpec((n, cb), lambda c, i: (0, c)),
                  pl.BlockSpec((1, cb), lambda c, i: (0, c))],
        out_specs=pl.BlockSpec((RB, cb), lambda c, i: (i, c)),
        out_shape=jax.ShapeDtypeStruct((n, w_), jnp.float32),
    )(idx, w, ef, bias.reshape(1, -1))


_SC_MESH = plsc.VectorSubcoreMesh(core_axis_name="c", subcore_axis_name="s")


def _sc_scatter_body(xl_hbm, idxs_hbm, ef_hbm, idx_v, xl_v, z_v, acc_sh,
                     ld_sem, sc_sem, wo_sem):
    c = lax.axis_index("c")
    s = lax.axis_index("s")
    pltpu.sync_copy(idxs_hbm.at[s], idx_v)

    def zrow(r, carry):
        for v in range(WC // 16):
            z_v[r, pl.ds(v * 16, 16)] = jnp.zeros((16,), jnp.float32)
        return carry
    lax.fori_loop(0, 128, zrow, 0)

    def xl_load(l, buf):
        g = c * CPC + l
        return pltpu.async_copy(
            xl_hbm.at[pl.ds(s * TPN, TPN), pl.ds(F_TC + g * WC, WC)],
            xl_v.at[buf], ld_sem)

    xl_load(0, 0).wait()
    wout = None
    for l in range(CPC):
        g = c * CPC + l
        b = l % 2
        if l + 1 < CPC:
            nxt = xl_load(l + 1, 1 - b)
        if wout is not None:
            wout.wait()
        z0 = pltpu.async_copy(z_v, acc_sh.at[pl.ds(s * TPN, 128)], wo_sem)
        z1 = pltpu.async_copy(z_v, acc_sh.at[pl.ds(s * TPN + 128, 128)],
                              wo_sem)

        @pl.when(s == 0)
        def _():
            pltpu.sync_copy(z_v, acc_sh.at[pl.ds(4096, 128)])

        z0.wait()
        z1.wait()
        plsc.subcore_barrier()
        descs = []
        for r in range(SLABS):
            half = r % 2
            descs.append(pltpu.async_copy(
                xl_v.at[b, pl.ds(half * 128, 128)],
                acc_sh.at[idx_v.at[r]], sc_sem, add=True))
        for d in descs:
            d.wait()
        plsc.subcore_barrier()
        wout = pltpu.async_copy(acc_sh.at[pl.ds(s * TPN, TPN)],
                                ef_hbm.at[g, pl.ds(s * TPN, TPN)], wo_sem)
        if l + 1 < CPC:
            nxt.wait()
    wout.wait()


_NBUF = 4


def _sc_gather_body(efs_hbm, idxs_hbm, out_hbm, idx_v, adj_v, lin_v, gbuf_v,
                    z_v, acc_sh, g_sem, a_sem, w_sem):
    c = lax.axis_index("c")
    s = lax.axis_index("s")
    pltpu.sync_copy(idxs_hbm.at[s], idx_v)
    for half in range(2):
        for v in range(8):
            lin_v[half, pl.ds(v * 16, 16)] = (
                s * TPN + half * 128 + v * 16 + lax.iota(jnp.int32, 16))

    def zrow(r, carry):
        for v in range(WC // 16):
            z_v[r, pl.ds(v * 16, 16)] = jnp.zeros((16,), jnp.float32)
        return carry
    lax.fori_loop(0, 128, zrow, 0)

    def adj(l):
        base = (c * CPC + l) * EF_ROWS

        def adjrow(r, carry):
            for v in range(8):
                adj_v[r, pl.ds(v * 16, 16)] = (
                    idx_v[r, pl.ds(v * 16, 16)] + base)
            return carry
        lax.fori_loop(0, SLABS, adjrow, 0)

    def gath(r):
        return pltpu.async_copy(efs_hbm.at[adj_v.at[r]],
                                gbuf_v.at[r % _NBUF], g_sem)

    wout = None
    for l in range(CPC):
        g = c * CPC + l
        adj(l)
        descs = [gath(r) for r in range(_NBUF)]
        if wout is not None:
            wout.wait()
        z0 = pltpu.async_copy(z_v, acc_sh.at[pl.ds(s * TPN, 128)], w_sem)
        z1 = pltpu.async_copy(z_v, acc_sh.at[pl.ds(s * TPN + 128, 128)],
                              w_sem)
        tail = []
        for r in range(SLABS):
            descs[r].wait()
            if r == 0:
                z0.wait()
                z1.wait()
            sc = pltpu.async_copy(gbuf_v.at[r % _NBUF],
                                  acc_sh.at[lin_v.at[r % 2]], a_sem, add=True)
            if r + _NBUF < SLABS:
                sc.wait()
                descs.append(gath(r + _NBUF))
            else:
                tail.append(sc)
        for sc in tail:
            sc.wait()
        wout = pltpu.async_copy(
            acc_sh.at[pl.ds(s * TPN, TPN)],
            out_hbm.at[pl.ds(s * TPN, TPN), pl.ds(g * WC, WC)], w_sem)
    wout.wait()


def _sc_scatter(xl1, idxs_slabs):
    n = xl1.shape[0]
    return pl.kernel(
        _sc_scatter_body,
        out_type=jax.ShapeDtypeStruct((NCH, EF_ROWS, WC), jnp.float32),
        mesh=_SC_MESH,
        scratch_types=[
            pltpu.VMEM((SLABS, 128), jnp.int32),
            pltpu.VMEM((2, TPN, WC), jnp.float32),
            pltpu.VMEM((128, WC), jnp.float32),
            pltpu.VMEM_SHARED((EF_ROWS, WC), jnp.float32),
            pltpu.SemaphoreType.DMA,
            pltpu.SemaphoreType.DMA,
            pltpu.SemaphoreType.DMA,
        ],
    )(xl1, idxs_slabs)


def _sc_gather(ef2d, idxs_slabs, n):
    return pl.kernel(
        _sc_gather_body,
        out_type=jax.ShapeDtypeStruct((n, NCH * WC), jnp.float32),
        mesh=_SC_MESH,
        scratch_types=[
            pltpu.VMEM((SLABS, 128), jnp.int32),
            pltpu.VMEM((SLABS, 128), jnp.int32),
            pltpu.VMEM((2, 128), jnp.int32),
            pltpu.VMEM((_NBUF, 128, WC), jnp.float32),
            pltpu.VMEM((128, WC), jnp.float32),
            pltpu.VMEM_SHARED((4096, WC), jnp.float32),
            pltpu.SemaphoreType.DMA,
            pltpu.SemaphoreType.DMA,
            pltpu.SemaphoreType.DMA,
        ],
    )(ef2d, idxs_slabs)


def kernel(x, theta1, bias1, theta2, bias2):
    n, d_in = x.shape
    hid = theta1.shape[1]
    code = theta2.shape[1]

    xn = pl.pallas_call(
        _normalize_body,
        grid=(n // RB,),
        in_specs=[pl.BlockSpec((RB, d_in), lambda i: (i, 0))],
        out_specs=pl.BlockSpec((RB, d_in), lambda i: (i, 0)),
        out_shape=jax.ShapeDtypeStruct((n, d_in), jnp.float32),
    )(x)

    idx, w, dinv8, idxsg = pl.pallas_call(
        _graph_body,
        grid=(n // RB,),
        in_specs=[pl.BlockSpec((RB, d_in), lambda i: (i, 0)),
                  pl.BlockSpec((n, d_in), lambda i: (0, 0))],
        out_specs=[pl.BlockSpec((RB, K), lambda i: (i, 0))] * 4,
        out_shape=[jax.ShapeDtypeStruct((n, K), jnp.int32),
                   jax.ShapeDtypeStruct((n, K), jnp.float32),
                   jax.ShapeDtypeStruct((n, K), jnp.float32),
                   jax.ShapeDtypeStruct((n, K), jnp.int32)],
    )(xn, xn)

    idxT = idx.T
    wT = w.T
    idxs_slabs = (idxsg.reshape(16, 2, 128, K)
                  .transpose(0, 3, 1, 2).reshape(16, SLABS, 128))

    xl1 = _matmul(x, theta1, 512, body=_matmul_bf16_body)

    binv_arr = pl.pallas_call(
        _binv_body,
        grid=(16,),
        in_specs=[pl.BlockSpec((K, n), lambda j: (0, 0)),
                  pl.BlockSpec((K, n), lambda j: (0, 0))],
        out_specs=pl.BlockSpec((JB, 128), lambda j: (j, 0)),
        out_shape=jax.ShapeDtypeStruct((EF_ROWS, 128), jnp.float32),
    )(idxT, wT)

    ef_raw = _sc_scatter(xl1, idxs_slabs)

    ef_s = pl.pallas_call(
        _scale_body,
        grid=(NCH,),
        in_specs=[pl.BlockSpec((1, EF_ROWS, WC), lambda g: (g, 0, 0)),
                  pl.BlockSpec((EF_ROWS, 128), lambda g: (0, 0))],
        out_specs=pl.BlockSpec((1, EF_ROWS, WC), lambda g: (g, 0, 0)),
        out_shape=jax.ShapeDtypeStruct((NCH, EF_ROWS, WC), jnp.float32),
    )(ef_raw, binv_arr)

    out1_raw = _sc_gather(ef_s.reshape(NCH * EF_ROWS, WC), idxs_slabs, n)

    xl2 = pl.pallas_call(
        _xl2_body,
        grid=(n // RB,),
        in_specs=[pl.BlockSpec((RB, hid), lambda i: (i, 0)),
                  pl.BlockSpec((RB, K), lambda i: (i, 0)),
                  pl.BlockSpec((1, hid), lambda i: (0, 0)),
                  pl.BlockSpec((hid, code), lambda i: (0, 0))],
        out_specs=pl.BlockSpec((RB, code), lambda i: (i, 0)),
        out_shape=jax.ShapeDtypeStruct((n, code), jnp.float32),
    )(out1_raw, dinv8, bias1.reshape(1, -1), theta2)

    ef2 = _edge_agg(idxT, wT, xl2, code)
    out2 = _node_agg(idx, w, ef2, bias2, code, jnp.tanh)
    return out2


# speedup vs baseline: 1.0664x; 1.0664x over previous
import functools

import jax
import jax.numpy as jnp
from jax import lax
from jax.experimental import pallas as pl
from jax.experimental.pallas import tpu as pltpu
from jax.experimental.pallas import tpu_sc as plsc

K = 8
EPS = 0.1
RB = 256
EF_ROWS = 4224
WC = 128
F_TC = 0
NCH = (4096 - F_TC) // WC
CPC = NCH // 2
TPN = 256
SLABS = 16
JB = EF_ROWS // 16


def _normalize_body(x_ref, o_ref):
    x = x_ref[...]
    nrm = jnp.sqrt(jnp.sum(x * x, axis=1, keepdims=True))
    o_ref[...] = x / jnp.maximum(nrm, 1e-12)


def _graph_body(xb_ref, xall_ref, idx_ref, w_ref, dinv_ref, idxsg_ref):
    n = xall_ref.shape[0]
    i0 = pl.program_id(0)
    s = lax.dot_general(xb_ref[...], xall_ref[...], (((1,), (1,)), ((), ())),
                        preferred_element_type=jnp.float32)
    cols = lax.broadcasted_iota(jnp.int32, (RB, n), 1)
    kcols = lax.broadcasted_iota(jnp.int32, (RB, K), 1)
    rows = lax.broadcasted_iota(jnp.int32, (RB, K), 0) + i0 * RB
    idx_out = jnp.zeros((RB, K), jnp.int32)
    w_out = jnp.zeros((RB, K), jnp.float32)
    for k in range(K):
        m = jnp.max(s, axis=1, keepdims=True)
        amin = jnp.min(jnp.where(s == m, cols, n), axis=1, keepdims=True)
        idx_out = jnp.where(kcols == k, amin, idx_out)
        w_out = jnp.where(kcols == k, (m > EPS).astype(jnp.float32), w_out)
        s = jnp.where(cols == amin, -jnp.inf, s)
    ddeg = jnp.sum(w_out, axis=1, keepdims=True)
    dinv = jnp.where(ddeg > 0, 1.0 / ddeg, 0.0)
    dummy = 4096 + (rows * K + kcols) % 128
    idx_ref[...] = idx_out
    w_ref[...] = w_out
    dinv_ref[...] = jnp.broadcast_to(dinv, (RB, K))
    idxsg_ref[...] = jnp.where(w_out > 0, idx_out, dummy)


def _matmul_body(a_ref, b_ref, o_ref):
    o_ref[...] = jnp.dot(a_ref[...], b_ref[...],
                         preferred_element_type=jnp.float32)


def _matmul_bf16_body(a_ref, b_ref, o_ref):
    a = a_ref[...].astype(jnp.bfloat16)
    b = b_ref[...].astype(jnp.bfloat16)
    o_ref[...] = jnp.dot(a, b, preferred_element_type=jnp.float32)


def _binv_body(idxT_ref, wT_ref, o_ref):
    j = pl.program_id(0)
    ji = j * JB + lax.broadcasted_iota(jnp.int32, (JB, 1), 0)
    bdeg = jnp.zeros((JB, 1), jnp.float32)
    for k in range(K):
        rowi = idxT_ref[k:k + 1, :]
        roww = wT_ref[k:k + 1, :]
        bdeg = bdeg + jnp.sum(jnp.where(rowi == ji, roww, 0.0),
                              axis=1, keepdims=True)
    binv = jnp.where(bdeg > 0, 1.0 / bdeg, 0.0)
    o_ref[...] = jnp.broadcast_to(binv, (JB, 128))


def _scale_body(ef_ref, binv_ref, o_ref):
    ef = ef_ref[...].reshape(EF_ROWS, WC)
    b = binv_ref[:, 0:1]
    o_ref[...] = jnp.where(b > 0, ef * b, 0.0).reshape(1, EF_ROWS, WC)


def _xl2_body(a_ref, dinv_ref, b1_ref, th2_ref, o_ref):
    a = jax.nn.relu(a_ref[...] * dinv_ref[:, 0:1] + b1_ref[...])
    o_ref[...] = jnp.dot(a, th2_ref[...], preferred_element_type=jnp.float32)


def _edge_body(idxT_ref, wT_ref, xl_ref, ef_ref):
    n = xl_ref.shape[0]
    j = pl.program_id(1)
    ji = j * RB + lax.broadcasted_iota(jnp.int32, (RB, 1), 0)
    ht = jnp.zeros((RB, n), jnp.float32)
    for k in range(K):
        rowi = idxT_ref[k:k + 1, :]
        roww = wT_ref[k:k + 1, :]
        ht = ht + jnp.where(rowi == ji, roww, 0.0)
    bdeg = jnp.sum(ht, axis=1, keepdims=True)
    binv = jnp.where(bdeg > 0, 1.0 / bdeg, 0.0)
    ef_ref[...] = jnp.dot(ht, xl_ref[...],
                          preferred_element_type=jnp.float32) * binv


def _node_body(idx_ref, w_ref, ef_ref, b_ref, o_ref, *, act):
    n = ef_ref.shape[0]
    cols = lax.broadcasted_iota(jnp.int32, (RB, n), 1)
    h = jnp.zeros((RB, n), jnp.float32)
    for k in range(K):
        h = h + jnp.where(idx_ref[:, k:k + 1] == cols, w_ref[:, k:k + 1], 0.0)
    ddeg = jnp.sum(w_ref[...], axis=1, keepdims=True)
    dinv = jnp.where(ddeg > 0, 1.0 / ddeg, 0.0)
    out = jnp.dot(h, ef_ref[...],
                  preferred_element_type=jnp.float32) * dinv + b_ref[...]
    o_ref[...] = act(out)


def _node_raw_body(idx_ref, w_ref, ef_ref, o_ref):
    n = ef_ref.shape[0]
    cols = lax.broadcasted_iota(jnp.int32, (RB, n), 1)
    h = jnp.zeros((RB, n), jnp.float32)
    for k in range(K):
        h = h + jnp.where(idx_ref[:, k:k + 1] == cols, w_ref[:, k:k + 1], 0.0)
    o_ref[...] = jnp.dot(h, ef_ref[...], preferred_element_type=jnp.float32)


def _node_raw(idx, w, ef, cb):
    n = ef.shape[0]
    w_ = ef.shape[1]
    return pl.pallas_call(
        _node_raw_body,
        grid=(w_ // cb, n // RB),
        in_specs=[pl.BlockSpec((RB, K), lambda c, i: (i, 0)),
                  pl.BlockSpec((RB, K), lambda c, i: (i, 0)),
                  pl.BlockSpec((n, cb), lambda c, i: (0, c))],
        out_specs=pl.BlockSpec((RB, cb), lambda c, i: (i, c)),
        out_shape=jax.ShapeDtypeStruct((n, w_), jnp.float32),
    )(idx, w, ef)


def _matmul_wide(a, b, body):
    m, kd = a.shape
    _, nd = b.shape
    return pl.pallas_call(
        body,
        grid=(m // RB,),
        in_specs=[pl.BlockSpec((RB, kd), lambda i: (i, 0)),
                  pl.BlockSpec((kd, nd), lambda i: (0, 0))],
        out_specs=pl.BlockSpec((RB, nd), lambda i: (i, 0)),
        out_shape=jax.ShapeDtypeStruct((m, nd), jnp.float32),
    )(a, b)


def _matmul(a, b, cb, body=_matmul_body):
    m, kd = a.shape
    _, nd = b.shape
    return pl.pallas_call(
        body,
        grid=(nd // cb, m // RB),
        in_specs=[pl.BlockSpec((RB, kd), lambda c, i: (i, 0)),
                  pl.BlockSpec((kd, cb), lambda c, i: (0, c))],
        out_specs=pl.BlockSpec((RB, cb), lambda c, i: (i, c)),
        out_shape=jax.ShapeDtypeStruct((m, nd), jnp.float32),
    )(a, b)


def _edge_agg(idxT, wT, xl, cb):
    n = xl.shape[0]
    w_ = xl.shape[1]
    return pl.pallas_call(
        _edge_body,
        grid=(w_ // cb, n // RB),
        in_specs=[pl.BlockSpec((K, n), lambda c, j: (0, 0)),
                  pl.BlockSpec((K, n), lambda c, j: (0, 0)),
                  pl.BlockSpec((n, cb), lambda c, j: (0, c))],
        out_specs=pl.BlockSpec((RB, cb), lambda c, j: (j, c)),
        out_shape=jax.ShapeDtypeStruct((n, w_), jnp.float32),
    )(idxT, wT, xl)


def _node_agg(idx, w, ef, bias, cb, act):
    n = ef.shape[0]
    w_ = ef.shape[1]
    return pl.pallas_call(
        functools.partial(_node_body, act=act),
        grid=(w_ // cb, n // RB),
        in_specs=[pl.BlockSpec((RB, K), lambda c, i: (i, 0)),
                  pl.BlockSpec((RB, K), lambda c, i: (i, 0)),
                  pl.BlockSpec((n, cb), lambda c, i: (0, c)),
                  pl.BlockSpec((1, cb), lambda c, i: (0, c))],
        out_specs=pl.BlockSpec((RB, cb), lambda c, i: (i, c)),
        out_shape=jax.ShapeDtypeStruct((n, w_), jnp.float32),
    )(idx, w, ef, bias.reshape(1, -1))


_SC_MESH = plsc.VectorSubcoreMesh(core_axis_name="c", subcore_axis_name="s")


def _sc_scatter_body(xl_hbm, idxs_hbm, ef_hbm, idx_v, xl_v, z_v, acc_sh,
                     ld_sem, sc_sem, wo_sem):
    c = lax.axis_index("c")
    s = lax.axis_index("s")
    pltpu.sync_copy(idxs_hbm.at[s], idx_v)

    def zrow(r, carry):
        for v in range(WC // 16):
            z_v[r, pl.ds(v * 16, 16)] = jnp.zeros((16,), jnp.float32)
        return carry
    lax.fori_loop(0, 128, zrow, 0)

    def xl_load(l, buf):
        g = c * CPC + l
        return pltpu.async_copy(
            xl_hbm.at[pl.ds(s * TPN, TPN), pl.ds(F_TC + g * WC, WC)],
            xl_v.at[buf], ld_sem)

    xl_load(0, 0).wait()
    wout = None
    for l in range(CPC):
        g = c * CPC + l
        b = l % 2
        if l + 1 < CPC:
            nxt = xl_load(l + 1, 1 - b)
        if wout is not None:
            wout.wait()
        z0 = pltpu.async_copy(z_v, acc_sh.at[pl.ds(s * TPN, 128)], wo_sem)
        z1 = pltpu.async_copy(z_v, acc_sh.at[pl.ds(s * TPN + 128, 128)],
                              wo_sem)

        @pl.when(s == 0)
        def _():
            pltpu.sync_copy(z_v, acc_sh.at[pl.ds(4096, 128)])

        z0.wait()
        z1.wait()
        plsc.subcore_barrier()
        descs = []
        for r in range(SLABS):
            half = r % 2
            descs.append(pltpu.async_copy(
                xl_v.at[b, pl.ds(half * 128, 128)],
                acc_sh.at[idx_v.at[r]], sc_sem, add=True))
        for d in descs:
            d.wait()
        plsc.subcore_barrier()
        wout = pltpu.async_copy(acc_sh.at[pl.ds(s * TPN, TPN)],
                                ef_hbm.at[g, pl.ds(s * TPN, TPN)], wo_sem)
        if l + 1 < CPC:
            nxt.wait()
    wout.wait()


_NBUF = 4


def _sc_gather_body(efs_hbm, idxs_hbm, out_hbm, idx_v, adj_v, lin_v, gbuf_v,
                    z_v, acc_sh, g_sem, a_sem, w_sem):
    c = lax.axis_index("c")
    s = lax.axis_index("s")
    pltpu.sync_copy(idxs_hbm.at[s], idx_v)
    for half in range(2):
        for v in range(8):
            lin_v[half, pl.ds(v * 16, 16)] = (
                s * TPN + half * 128 + v * 16 + lax.iota(jnp.int32, 16))

    def zrow(r, carry):
        for v in range(WC // 16):
            z_v[r, pl.ds(v * 16, 16)] = jnp.zeros((16,), jnp.float32)
        return carry
    lax.fori_loop(0, 128, zrow, 0)

    def adj(l):
        base = (c * CPC + l) * EF_ROWS

        def adjrow(r, carry):
            for v in range(8):
                adj_v[r, pl.ds(v * 16, 16)] = (
                    idx_v[r, pl.ds(v * 16, 16)] + base)
            return carry
        lax.fori_loop(0, SLABS, adjrow, 0)

    def gath(r):
        return pltpu.async_copy(efs_hbm.at[adj_v.at[r]],
                                gbuf_v.at[r % _NBUF], g_sem)

    wout = None
    for l in range(CPC):
        g = c * CPC + l
        adj(l)
        descs = [gath(r) for r in range(_NBUF)]
        if wout is not None:
            wout.wait()
        z0 = pltpu.async_copy(z_v, acc_sh.at[pl.ds(s * TPN, 128)], w_sem)
        z1 = pltpu.async_copy(z_v, acc_sh.at[pl.ds(s * TPN + 128, 128)],
                              w_sem)
        tail = []
        for r in range(SLABS):
            descs[r].wait()
            if r == 0:
                z0.wait()
                z1.wait()
            sc = pltpu.async_copy(gbuf_v.at[r % _NBUF],
                                  acc_sh.at[lin_v.at[r % 2]], a_sem, add=True)
            if r + _NBUF < SLABS:
                sc.wait()
                descs.append(gath(r + _NBUF))
            else:
                tail.append(sc)
        for sc in tail:
            sc.wait()
        wout = pltpu.async_copy(
            acc_sh.at[pl.ds(s * TPN, TPN)],
            out_hbm.at[pl.ds(s * TPN, TPN), pl.ds(g * WC, WC)], w_sem)
    wout.wait()


def _sc_scatter(xl1, idxs_slabs):
    n = xl1.shape[0]
    return pl.kernel(
        _sc_scatter_body,
        out_type=jax.ShapeDtypeStruct((NCH, EF_ROWS, WC), jnp.float32),
        mesh=_SC_MESH,
        scratch_types=[
            pltpu.VMEM((SLABS, 128), jnp.int32),
            pltpu.VMEM((2, TPN, WC), jnp.float32),
            pltpu.VMEM((128, WC), jnp.float32),
            pltpu.VMEM_SHARED((EF_ROWS, WC), jnp.float32),
            pltpu.SemaphoreType.DMA,
            pltpu.SemaphoreType.DMA,
            pltpu.SemaphoreType.DMA,
        ],
    )(xl1, idxs_slabs)


def _sc_gather(ef2d, idxs_slabs, n):
    return pl.kernel(
        _sc_gather_body,
        out_type=jax.ShapeDtypeStruct((n, NCH * WC), jnp.float32),
        mesh=_SC_MESH,
        scratch_types=[
            pltpu.VMEM((SLABS, 128), jnp.int32),
            pltpu.VMEM((SLABS, 128), jnp.int32),
            pltpu.VMEM((2, 128), jnp.int32),
            pltpu.VMEM((_NBUF, 128, WC), jnp.float32),
            pltpu.VMEM((128, WC), jnp.float32),
            pltpu.VMEM_SHARED((4096, WC), jnp.float32),
            pltpu.SemaphoreType.DMA,
            pltpu.SemaphoreType.DMA,
            pltpu.SemaphoreType.DMA,
        ],
    )(ef2d, idxs_slabs)


def kernel(x, theta1, bias1, theta2, bias2):
    n, d_in = x.shape
    hid = theta1.shape[1]
    code = theta2.shape[1]

    xn = pl.pallas_call(
        _normalize_body,
        grid=(n // RB,),
        in_specs=[pl.BlockSpec((RB, d_in), lambda i: (i, 0))],
        out_specs=pl.BlockSpec((RB, d_in), lambda i: (i, 0)),
        out_shape=jax.ShapeDtypeStruct((n, d_in), jnp.float32),
    )(x)

    idx, w, dinv8, idxsg = pl.pallas_call(
        _graph_body,
        grid=(n // RB,),
        in_specs=[pl.BlockSpec((RB, d_in), lambda i: (i, 0)),
                  pl.BlockSpec((n, d_in), lambda i: (0, 0))],
        out_specs=[pl.BlockSpec((RB, K), lambda i: (i, 0))] * 4,
        out_shape=[jax.ShapeDtypeStruct((n, K), jnp.int32),
                   jax.ShapeDtypeStruct((n, K), jnp.float32),
                   jax.ShapeDtypeStruct((n, K), jnp.float32),
                   jax.ShapeDtypeStruct((n, K), jnp.int32)],
    )(xn, xn)

    idxT = idx.T
    wT = w.T
    idxs_slabs = (idxsg.reshape(16, 2, 128, K)
                  .transpose(0, 3, 1, 2).reshape(16, SLABS, 128))

    xl1 = _matmul_wide(x, theta1, _matmul_bf16_body)

    binv_arr = pl.pallas_call(
        _binv_body,
        grid=(16,),
        in_specs=[pl.BlockSpec((K, n), lambda j: (0, 0)),
                  pl.BlockSpec((K, n), lambda j: (0, 0))],
        out_specs=pl.BlockSpec((JB, 128), lambda j: (j, 0)),
        out_shape=jax.ShapeDtypeStruct((EF_ROWS, 128), jnp.float32),
    )(idxT, wT)

    ef_raw = _sc_scatter(xl1, idxs_slabs)

    ef_s = pl.pallas_call(
        _scale_body,
        grid=(NCH,),
        in_specs=[pl.BlockSpec((1, EF_ROWS, WC), lambda g: (g, 0, 0)),
                  pl.BlockSpec((EF_ROWS, 128), lambda g: (0, 0))],
        out_specs=pl.BlockSpec((1, EF_ROWS, WC), lambda g: (g, 0, 0)),
        out_shape=jax.ShapeDtypeStruct((NCH, EF_ROWS, WC), jnp.float32),
    )(ef_raw, binv_arr)

    out1_raw = _sc_gather(ef_s.reshape(NCH * EF_ROWS, WC), idxs_slabs, n)

    xl2 = pl.pallas_call(
        _xl2_body,
        grid=(n // RB,),
        in_specs=[pl.BlockSpec((RB, hid), lambda i: (i, 0)),
                  pl.BlockSpec((RB, K), lambda i: (i, 0)),
                  pl.BlockSpec((1, hid), lambda i: (0, 0)),
                  pl.BlockSpec((hid, code), lambda i: (0, 0))],
        out_specs=pl.BlockSpec((RB, code), lambda i: (i, 0)),
        out_shape=jax.ShapeDtypeStruct((n, code), jnp.float32),
    )(out1_raw, dinv8, bias1.reshape(1, -1), theta2)

    ef2 = _edge_agg(idxT, wT, xl2, code)
    out2 = _node_agg(idx, w, ef2, bias2, code, jnp.tanh)
    return out2


# speedup vs baseline: 1.0671x; 1.0006x over previous
import functools

import jax
import jax.numpy as jnp
from jax import lax
from jax.experimental import pallas as pl
from jax.experimental.pallas import tpu as pltpu
from jax.experimental.pallas import tpu_sc as plsc

K = 8
EPS = 0.1
RB = 256
EF_ROWS = 4224
WC = 128
F_TC = 0
NCH = (4096 - F_TC) // WC
CPC = NCH // 2
TPN = 256
SLABS = 16
JB = EF_ROWS // 16


def _normalize_body(x_ref, o_ref):
    x = x_ref[...]
    nrm = jnp.sqrt(jnp.sum(x * x, axis=1, keepdims=True))
    o_ref[...] = x / jnp.maximum(nrm, 1e-12)


def _graph_body(xb_ref, xall_ref, idx_ref, w_ref, dinv_ref, idxsg_ref):
    n = xall_ref.shape[0]
    i0 = pl.program_id(0)
    s = lax.dot_general(xb_ref[...], xall_ref[...], (((1,), (1,)), ((), ())),
                        preferred_element_type=jnp.float32)
    cols = lax.broadcasted_iota(jnp.int32, (RB, n), 1)
    kcols = lax.broadcasted_iota(jnp.int32, (RB, K), 1)
    rows = lax.broadcasted_iota(jnp.int32, (RB, K), 0) + i0 * RB
    idx_out = jnp.zeros((RB, K), jnp.int32)
    w_out = jnp.zeros((RB, K), jnp.float32)
    for k in range(K):
        m = jnp.max(s, axis=1, keepdims=True)
        amin = jnp.min(jnp.where(s == m, cols, n), axis=1, keepdims=True)
        idx_out = jnp.where(kcols == k, amin, idx_out)
        w_out = jnp.where(kcols == k, (m > EPS).astype(jnp.float32), w_out)
        s = jnp.where(cols == amin, -jnp.inf, s)
    ddeg = jnp.sum(w_out, axis=1, keepdims=True)
    dinv = jnp.where(ddeg > 0, 1.0 / ddeg, 0.0)
    dummy = 4096 + (rows * K + kcols) % 128
    idx_ref[...] = idx_out
    w_ref[...] = w_out
    dinv_ref[...] = jnp.broadcast_to(dinv, (RB, K))
    idxsg_ref[...] = jnp.where(w_out > 0, idx_out, dummy)


def _matmul_body(a_ref, b_ref, o_ref):
    o_ref[...] = jnp.dot(a_ref[...], b_ref[...],
                         preferred_element_type=jnp.float32)


def _matmul_bf16_body(a_ref, b_ref, o_ref):
    a = a_ref[...].astype(jnp.bfloat16)
    b = b_ref[...].astype(jnp.bfloat16)
    o_ref[...] = jnp.dot(a, b, preferred_element_type=jnp.float32)


def _binv_body(idxT_ref, wT_ref, o_ref):
    j = pl.program_id(0)
    ji = j * JB + lax.broadcasted_iota(jnp.int32, (JB, 1), 0)
    bdeg = jnp.zeros((JB, 1), jnp.float32)
    for k in range(K):
        rowi = idxT_ref[k:k + 1, :]
        roww = wT_ref[k:k + 1, :]
        bdeg = bdeg + jnp.sum(jnp.where(rowi == ji, roww, 0.0),
                              axis=1, keepdims=True)
    binv = jnp.where(bdeg > 0, 1.0 / bdeg, 0.0)
    o_ref[...] = jnp.broadcast_to(binv, (JB, 128))


def _scale_body(ef_ref, binv_ref, o_ref):
    ef = ef_ref[...].reshape(EF_ROWS, WC)
    b = binv_ref[:, 0:1]
    o_ref[...] = jnp.where(b > 0, ef * b, 0.0).reshape(1, EF_ROWS, WC)


def _xl2_body(a_ref, dinv_ref, b1_ref, th2_ref, o_ref):
    a = jax.nn.relu(a_ref[...] * dinv_ref[:, 0:1] + b1_ref[...])
    o_ref[...] = jnp.dot(a, th2_ref[...], preferred_element_type=jnp.float32)


def _edge_body(idxT_ref, wT_ref, xl_ref, ef_ref):
    n = xl_ref.shape[0]
    j = pl.program_id(1)
    ji = j * RB + lax.broadcasted_iota(jnp.int32, (RB, 1), 0)
    ht = jnp.zeros((RB, n), jnp.float32)
    for k in range(K):
        rowi = idxT_ref[k:k + 1, :]
        roww = wT_ref[k:k + 1, :]
        ht = ht + jnp.where(rowi == ji, roww, 0.0)
    bdeg = jnp.sum(ht, axis=1, keepdims=True)
    binv = jnp.where(bdeg > 0, 1.0 / bdeg, 0.0)
    ef_ref[...] = jnp.dot(ht, xl_ref[...],
                          preferred_element_type=jnp.float32) * binv


def _node_body(idx_ref, w_ref, ef_ref, b_ref, o_ref, *, act):
    n = ef_ref.shape[0]
    cols = lax.broadcasted_iota(jnp.int32, (RB, n), 1)
    h = jnp.zeros((RB, n), jnp.float32)
    for k in range(K):
        h = h + jnp.where(idx_ref[:, k:k + 1] == cols, w_ref[:, k:k + 1], 0.0)
    ddeg = jnp.sum(w_ref[...], axis=1, keepdims=True)
    dinv = jnp.where(ddeg > 0, 1.0 / ddeg, 0.0)
    out = jnp.dot(h, ef_ref[...],
                  preferred_element_type=jnp.float32) * dinv + b_ref[...]
    o_ref[...] = act(out)


def _matmul_wide(a, b, body):
    m, kd = a.shape
    _, nd = b.shape
    return pl.pallas_call(
        body,
        grid=(m // RB,),
        in_specs=[pl.BlockSpec((RB, kd), lambda i: (i, 0)),
                  pl.BlockSpec((kd, nd), lambda i: (0, 0))],
        out_specs=pl.BlockSpec((RB, nd), lambda i: (i, 0)),
        out_shape=jax.ShapeDtypeStruct((m, nd), jnp.float32),
    )(a, b)


def _matmul(a, b, cb, body=_matmul_body):
    m, kd = a.shape
    _, nd = b.shape
    return pl.pallas_call(
        body,
        grid=(nd // cb, m // RB),
        in_specs=[pl.BlockSpec((RB, kd), lambda c, i: (i, 0)),
                  pl.BlockSpec((kd, cb), lambda c, i: (0, c))],
        out_specs=pl.BlockSpec((RB, cb), lambda c, i: (i, c)),
        out_shape=jax.ShapeDtypeStruct((m, nd), jnp.float32),
    )(a, b)


def _edge_agg(idxT, wT, xl, cb):
    n = xl.shape[0]
    w_ = xl.shape[1]
    return pl.pallas_call(
        _edge_body,
        grid=(w_ // cb, n // RB),
        in_specs=[pl.BlockSpec((K, n), lambda c, j: (0, 0)),
                  pl.BlockSpec((K, n), lambda c, j: (0, 0)),
                  pl.BlockSpec((n, cb), lambda c, j: (0, c))],
        out_specs=pl.BlockSpec((RB, cb), lambda c, j: (j, c)),
        out_shape=jax.ShapeDtypeStruct((n, w_), jnp.float32),
    )(idxT, wT, xl)


def _node_agg(idx, w, ef, bias, cb, act):
    n = ef.shape[0]
    w_ = ef.shape[1]
    return pl.pallas_call(
        functools.partial(_node_body, act=act),
        grid=(w_ // cb, n // RB),
        in_specs=[pl.BlockSpec((RB, K), lambda c, i: (i, 0)),
                  pl.BlockSpec((RB, K), lambda c, i: (i, 0)),
                  pl.BlockSpec((n, cb), lambda c, i: (0, c)),
                  pl.BlockSpec((1, cb), lambda c, i: (0, c))],
        out_specs=pl.BlockSpec((RB, cb), lambda c, i: (i, c)),
        out_shape=jax.ShapeDtypeStruct((n, w_), jnp.float32),
    )(idx, w, ef, bias.reshape(1, -1))


_SC_MESH = plsc.VectorSubcoreMesh(core_axis_name="c", subcore_axis_name="s")


def _sc_scatter_body(xl_hbm, idxs_hbm, ef_hbm, idx_v, xl_v, z_v, acc_sh,
                     ld_sem, sc_sem, wo_sem):
    c = lax.axis_index("c")
    s = lax.axis_index("s")
    pltpu.sync_copy(idxs_hbm.at[s], idx_v)

    def zrow(r, carry):
        for v in range(WC // 16):
            z_v[r, pl.ds(v * 16, 16)] = jnp.zeros((16,), jnp.float32)
        return carry
    lax.fori_loop(0, 128, zrow, 0)

    def xl_load(l, buf):
        g = c * CPC + l
        return pltpu.async_copy(
            xl_hbm.at[pl.ds(s * TPN, TPN), pl.ds(F_TC + g * WC, WC)],
            xl_v.at[buf], ld_sem)

    xl_load(0, 0).wait()
    wout = None
    for l in range(CPC):
        g = c * CPC + l
        b = l % 2
        if l + 1 < CPC:
            nxt = xl_load(l + 1, 1 - b)
        if wout is not None:
            wout.wait()
        z0 = pltpu.async_copy(z_v, acc_sh.at[pl.ds(s * TPN, 128)], wo_sem)
        z1 = pltpu.async_copy(z_v, acc_sh.at[pl.ds(s * TPN + 128, 128)],
                              wo_sem)

        @pl.when(s == 0)
        def _():
            pltpu.sync_copy(z_v, acc_sh.at[pl.ds(4096, 128)])

        z0.wait()
        z1.wait()
        plsc.subcore_barrier()
        descs = []
        for r in range(SLABS):
            half = r % 2
            descs.append(pltpu.async_copy(
                xl_v.at[b, pl.ds(half * 128, 128)],
                acc_sh.at[idx_v.at[r]], sc_sem, add=True))
        for d in descs:
            d.wait()
        plsc.subcore_barrier()
        wout = pltpu.async_copy(acc_sh.at[pl.ds(s * TPN, TPN)],
                                ef_hbm.at[g, pl.ds(s * TPN, TPN)], wo_sem)
        if l + 1 < CPC:
            nxt.wait()
    wout.wait()


_NBUF = 4


def _sc_gather_body(efs_hbm, idxs_hbm, out_hbm, idx_v, adj_v, lin_v, gbuf_v,
                    z_v, acc_sh, g_sem, a_sem, w_sem):
    c = lax.axis_index("c")
    s = lax.axis_index("s")
    pltpu.sync_copy(idxs_hbm.at[s], idx_v)
    for half in range(2):
        for v in range(8):
            lin_v[half, pl.ds(v * 16, 16)] = (
                s * TPN + half * 128 + v * 16 + lax.iota(jnp.int32, 16))

    def zrow(r, carry):
        for v in range(WC // 16):
            z_v[r, pl.ds(v * 16, 16)] = jnp.zeros((16,), jnp.float32)
        return carry
    lax.fori_loop(0, 128, zrow, 0)

    def adj(l):
        base = (c * CPC + l) * EF_ROWS

        def adjrow(r, carry):
            for v in range(8):
                adj_v[r, pl.ds(v * 16, 16)] = (
                    idx_v[r, pl.ds(v * 16, 16)] + base)
            return carry
        lax.fori_loop(0, SLABS, adjrow, 0)

    def gath(r):
        return pltpu.async_copy(efs_hbm.at[adj_v.at[r]],
                                gbuf_v.at[r % _NBUF], g_sem)

    wout = None
    for l in range(CPC):
        g = c * CPC + l
        adj(l)
        descs = [gath(r) for r in range(_NBUF)]
        if wout is not None:
            wout.wait()
        z0 = pltpu.async_copy(z_v, acc_sh.at[pl.ds(s * TPN, 128)], w_sem)
        z1 = pltpu.async_copy(z_v, acc_sh.at[pl.ds(s * TPN + 128, 128)],
                              w_sem)
        tail = []
        for r in range(SLABS):
            descs[r].wait()
            if r == 0:
                z0.wait()
                z1.wait()
            sc = pltpu.async_copy(gbuf_v.at[r % _NBUF],
                                  acc_sh.at[lin_v.at[r % 2]], a_sem, add=True)
            if r + _NBUF < SLABS:
                sc.wait()
                descs.append(gath(r + _NBUF))
            else:
                tail.append(sc)
        for sc in tail:
            sc.wait()
        wout = pltpu.async_copy(
            acc_sh.at[pl.ds(s * TPN, TPN)],
            out_hbm.at[pl.ds(s * TPN, TPN), pl.ds(g * WC, WC)], w_sem)
    wout.wait()


def _sc_scatter(xl1, idxs_slabs):
    n = xl1.shape[0]
    return pl.kernel(
        _sc_scatter_body,
        out_type=jax.ShapeDtypeStruct((NCH, EF_ROWS, WC), jnp.float32),
        mesh=_SC_MESH,
        scratch_types=[
            pltpu.VMEM((SLABS, 128), jnp.int32),
            pltpu.VMEM((2, TPN, WC), jnp.float32),
            pltpu.VMEM((128, WC), jnp.float32),
            pltpu.VMEM_SHARED((EF_ROWS, WC), jnp.float32),
            pltpu.SemaphoreType.DMA,
            pltpu.SemaphoreType.DMA,
            pltpu.SemaphoreType.DMA,
        ],
    )(xl1, idxs_slabs)


def _sc_gather(ef2d, idxs_slabs, n):
    return pl.kernel(
        _sc_gather_body,
        out_type=jax.ShapeDtypeStruct((n, NCH * WC), jnp.float32),
        mesh=_SC_MESH,
        scratch_types=[
            pltpu.VMEM((SLABS, 128), jnp.int32),
            pltpu.VMEM((SLABS, 128), jnp.int32),
            pltpu.VMEM((2, 128), jnp.int32),
            pltpu.VMEM((_NBUF, 128, WC), jnp.float32),
            pltpu.VMEM((128, WC), jnp.float32),
            pltpu.VMEM_SHARED((4096, WC), jnp.float32),
            pltpu.SemaphoreType.DMA,
            pltpu.SemaphoreType.DMA,
            pltpu.SemaphoreType.DMA,
        ],
    )(ef2d, idxs_slabs)


def kernel(x, theta1, bias1, theta2, bias2):
    n, d_in = x.shape
    hid = theta1.shape[1]
    code = theta2.shape[1]

    xn = pl.pallas_call(
        _normalize_body,
        grid=(n // RB,),
        in_specs=[pl.BlockSpec((RB, d_in), lambda i: (i, 0))],
        out_specs=pl.BlockSpec((RB, d_in), lambda i: (i, 0)),
        out_shape=jax.ShapeDtypeStruct((n, d_in), jnp.float32),
    )(x)

    idx, w, dinv8, idxsg = pl.pallas_call(
        _graph_body,
        grid=(n // RB,),
        in_specs=[pl.BlockSpec((RB, d_in), lambda i: (i, 0)),
                  pl.BlockSpec((n, d_in), lambda i: (0, 0))],
        out_specs=[pl.BlockSpec((RB, K), lambda i: (i, 0))] * 4,
        out_shape=[jax.ShapeDtypeStruct((n, K), jnp.int32),
                   jax.ShapeDtypeStruct((n, K), jnp.float32),
                   jax.ShapeDtypeStruct((n, K), jnp.float32),
                   jax.ShapeDtypeStruct((n, K), jnp.int32)],
    )(xn, xn)

    idxT = idx.T
    wT = w.T
    idxs_slabs = (idxsg.reshape(16, 2, 128, K)
                  .transpose(0, 3, 1, 2).reshape(16, SLABS, 128))

    xl1 = _matmul_wide(x, theta1, _matmul_bf16_body)

    binv_arr = pl.pallas_call(
        _binv_body,
        grid=(16,),
        in_specs=[pl.BlockSpec((K, n), lambda j: (0, 0)),
                  pl.BlockSpec((K, n), lambda j: (0, 0))],
        out_specs=pl.BlockSpec((JB, 128), lambda j: (j, 0)),
        out_shape=jax.ShapeDtypeStruct((EF_ROWS, 128), jnp.float32),
    )(idxT, wT)

    ef_raw = _sc_scatter(xl1, idxs_slabs)

    ef_s = pl.pallas_call(
        _scale_body,
        grid=(NCH,),
        in_specs=[pl.BlockSpec((1, EF_ROWS, WC), lambda g: (g, 0, 0)),
                  pl.BlockSpec((EF_ROWS, 128), lambda g: (0, 0))],
        out_specs=pl.BlockSpec((1, EF_ROWS, WC), lambda g: (g, 0, 0)),
        out_shape=jax.ShapeDtypeStruct((NCH, EF_ROWS, WC), jnp.float32),
    )(ef_raw, binv_arr)

    out1_raw = _sc_gather(ef_s.reshape(NCH * EF_ROWS, WC), idxs_slabs, n)

    xl2 = pl.pallas_call(
        _xl2_body,
        grid=(n // RB,),
        in_specs=[pl.BlockSpec((RB, hid), lambda i: (i, 0)),
                  pl.BlockSpec((RB, K), lambda i: (i, 0)),
                  pl.BlockSpec((1, hid), lambda i: (0, 0)),
                  pl.BlockSpec((hid, code), lambda i: (0, 0))],
        out_specs=pl.BlockSpec((RB, code), lambda i: (i, 0)),
        out_shape=jax.ShapeDtypeStruct((n, code), jnp.float32),
    )(out1_raw, dinv8, bias1.reshape(1, -1), theta2)

    ef2 = _edge_agg(idxT, wT, xl2, code)
    out2 = _node_agg(idx, w, ef2, bias2, code, jnp.tanh)
    return out2
